# baseline (speedup 1.0000x reference)
import jax
import jax.numpy as jnp
from jax.experimental import pallas as pl

N = 5000
NP = 5120
CB = 256
K = 1500
SEL_CH = 500


def _counts_kernel(src_all, tgt_all, srcT_blk, tgtT_blk,
                   sqs_col, sqt_col, sqs_row, sqt_row, out_ref):
    def dist(pts, ptsT, sq_col, sq_row):
        g = jax.lax.dot_general(pts, ptsT, (((1,), (0,)), ((), ())),
                                preferred_element_type=jnp.float32)
        d2 = (sq_col + sq_row) - 2.0 * g
        return jnp.sqrt(jnp.maximum(d2, 1e-12))

    ds = dist(src_all[...], srcT_blk[...], sqs_col[...], sqs_row[...])
    dt = dist(tgt_all[...], tgtT_blk[...], sqt_col[...], sqt_row[...])
    delta = jnp.abs(ds - dt)
    sc = jnp.maximum(1.0 - (delta ** 2) / (0.1 ** 2), 0.0)

    total = jnp.zeros((1, CB), jnp.float32)
    for k in range(8):
        w = sc[640 * k:640 * (k + 1), :]
        acc = w[0:8, :]
        for c in range(1, 80):
            acc = acc + w[8 * c:8 * c + 8, :]
        p = acc[0:4, :] + acc[4:8, :]
        q = p[0:2, :] + p[2:4, :]
        r = q[0:1, :] + q[1:2, :]
        total = total + r
    out_ref[...] = total


_counts_call = pl.pallas_call(
    _counts_kernel,
    grid=(NP // CB,),
    in_specs=[pl.BlockSpec((NP, 3), lambda b: (0, 0)),
              pl.BlockSpec((NP, 3), lambda b: (0, 0)),
              pl.BlockSpec((3, CB), lambda b: (0, b)),
              pl.BlockSpec((3, CB), lambda b: (0, b)),
              pl.BlockSpec((NP, 1), lambda b: (0, 0)),
              pl.BlockSpec((NP, 1), lambda b: (0, 0)),
              pl.BlockSpec((1, CB), lambda b: (0, b)),
              pl.BlockSpec((1, CB), lambda b: (0, b))],
    out_specs=pl.BlockSpec((1, CB), lambda b: (0, b)),
    out_shape=jax.ShapeDtypeStruct((1, NP), jnp.float32),
)


def _select_kernel(ccol_ref, crow_ref, out_ref):
    crow = crow_ref[...]
    jidx = jax.lax.broadcasted_iota(jnp.int32, (1, N), 1)
    kio = jax.lax.broadcasted_iota(jnp.int32, (1, K), 1)
    acc = jnp.zeros((1, K), jnp.int32)
    for c0 in range(0, N, SEL_CH):
        ci = ccol_ref[pl.ds(c0, SEL_CH), :]
        iidx = jax.lax.broadcasted_iota(jnp.int32, (SEL_CH, 1), 0) + c0
        beat = (crow > ci) | ((crow == ci) & (jidx < iidx))
        rank = jnp.sum(beat.astype(jnp.int32), axis=1)[:, None]
        hit = (rank == kio).astype(jnp.int32)
        acc = acc + jnp.sum(hit * iidx, axis=0)[None, :]
    out_ref[...] = acc


_select_call = pl.pallas_call(
    _select_kernel,
    in_specs=[pl.BlockSpec((N, 1), lambda: (0, 0)),
              pl.BlockSpec((1, N), lambda: (0, 0))],
    out_specs=pl.BlockSpec((1, K), lambda: (0, 0)),
    out_shape=jax.ShapeDtypeStruct((1, K), jnp.int32),
)


def kernel(src_corr_points, tgt_corr_points):
    src = src_corr_points.astype(jnp.float32)
    tgt = tgt_corr_points.astype(jnp.float32)
    pad_s = jnp.full((NP - N, 3), 1e6, jnp.float32)
    pad_t = jnp.full((NP - N, 3), 3e6, jnp.float32)
    srcp = jnp.concatenate([src, pad_s], axis=0)
    tgtp = jnp.concatenate([tgt, pad_t], axis=0)
    sqs = jnp.sum(srcp * srcp, axis=-1)
    sqt = jnp.sum(tgtp * tgtp, axis=-1)

    counts = _counts_call(srcp, tgtp, srcp.T, tgtp.T,
                          sqs[:, None], sqt[:, None],
                          sqs[None, :], sqt[None, :])
    counts = counts[:, 0:N]
    idx = _select_call(counts.T, counts)
    return idx[0]


# speedup vs baseline: 1.3674x; 1.3674x over previous
import jax
import jax.numpy as jnp
from jax.experimental import pallas as pl

N = 5000
NP = 5120
CB = 512
K = 1500
SEL_CH = 500


def _counts_kernel(src_all, tgt_all, srcT_blk, tgtT_blk,
                   sqs_col, sqt_col, sqs_row, sqt_row, out_ref):
    def dist(pts, ptsT, sq_col, sq_row):
        g = jax.lax.dot_general(pts, ptsT, (((1,), (0,)), ((), ())),
                                preferred_element_type=jnp.float32)
        d2 = (sq_col + sq_row) - 2.0 * g
        a = jnp.maximum(d2, 1e-12)
        return a * jax.lax.rsqrt(a)

    ds = dist(src_all[...], srcT_blk[...], sqs_col[...], sqs_row[...])
    dt = dist(tgt_all[...], tgtT_blk[...], sqt_col[...], sqt_row[...])
    delta = jnp.abs(ds - dt)
    sc = jnp.maximum(1.0 - (delta ** 2) / (0.1 ** 2), 0.0)

    total = jnp.zeros((1, CB), jnp.float32)
    for k in range(8):
        w = sc[640 * k:640 * (k + 1), :]
        acc = w[0:8, :]
        for c in range(1, 80):
            acc = acc + w[8 * c:8 * c + 8, :]
        p = acc[0:4, :] + acc[4:8, :]
        q = p[0:2, :] + p[2:4, :]
        r = q[0:1, :] + q[1:2, :]
        total = total + r
    out_ref[...] = total


_counts_call = pl.pallas_call(
    _counts_kernel,
    grid=(NP // CB,),
    in_specs=[pl.BlockSpec((NP, 3), lambda b: (0, 0)),
              pl.BlockSpec((NP, 3), lambda b: (0, 0)),
              pl.BlockSpec((3, CB), lambda b: (0, b)),
              pl.BlockSpec((3, CB), lambda b: (0, b)),
              pl.BlockSpec((NP, 1), lambda b: (0, 0)),
              pl.BlockSpec((NP, 1), lambda b: (0, 0)),
              pl.BlockSpec((1, CB), lambda b: (0, b)),
              pl.BlockSpec((1, CB), lambda b: (0, b))],
    out_specs=pl.BlockSpec((1, CB), lambda b: (0, b)),
    out_shape=jax.ShapeDtypeStruct((1, NP), jnp.float32),
)


def _select_kernel(ccol_ref, crow_ref, out_ref):
    crow = crow_ref[...]
    jidx = jax.lax.broadcasted_iota(jnp.int32, (1, N), 1)
    kio = jax.lax.broadcasted_iota(jnp.int32, (1, K), 1)
    acc = jnp.zeros((1, K), jnp.int32)
    for c0 in range(0, N, SEL_CH):
        ci = ccol_ref[pl.ds(c0, SEL_CH), :]
        iidx = jax.lax.broadcasted_iota(jnp.int32, (SEL_CH, 1), 0) + c0
        beat = (crow > ci) | ((crow == ci) & (jidx < iidx))
        rank = jnp.sum(beat.astype(jnp.int32), axis=1)[:, None]
        hit = (rank == kio).astype(jnp.int32)
        acc = acc + jnp.sum(hit * iidx, axis=0)[None, :]
    out_ref[...] = acc


_select_call = pl.pallas_call(
    _select_kernel,
    in_specs=[pl.BlockSpec((N, 1), lambda: (0, 0)),
              pl.BlockSpec((1, N), lambda: (0, 0))],
    out_specs=pl.BlockSpec((1, K), lambda: (0, 0)),
    out_shape=jax.ShapeDtypeStruct((1, K), jnp.int32),
)


def kernel(src_corr_points, tgt_corr_points):
    src = src_corr_points.astype(jnp.float32)
    tgt = tgt_corr_points.astype(jnp.float32)
    pad_s = jnp.full((NP - N, 3), 1e6, jnp.float32)
    pad_t = jnp.full((NP - N, 3), 3e6, jnp.float32)
    srcp = jnp.concatenate([src, pad_s], axis=0)
    tgtp = jnp.concatenate([tgt, pad_t], axis=0)
    sqs = jnp.sum(srcp * srcp, axis=-1)
    sqt = jnp.sum(tgtp * tgtp, axis=-1)

    counts = _counts_call(srcp, tgtp, srcp.T, tgtp.T,
                          sqs[:, None], sqt[:, None],
                          sqs[None, :], sqt[None, :])
    counts = counts[:, 0:N]
    idx = _select_call(counts.T, counts)
    return idx[0]


# speedup vs baseline: 1.4247x; 1.0419x over previous
import jax
import jax.numpy as jnp
from jax.experimental import pallas as pl

N = 5000
NP = 5120
CB = 512
K = 1500
SEL_CH = 500


def _counts_kernel(src_all, tgt_all, srcT_blk, tgtT_blk,
                   sqs_col, sqt_col, sqs_row, sqt_row, out_ref):
    def dist(pts, ptsT, sq_col, sq_row):
        g = jax.lax.dot_general(pts, ptsT, (((1,), (0,)), ((), ())),
                                preferred_element_type=jnp.float32)
        d2 = (sq_col + sq_row) - 2.0 * g
        a = jnp.maximum(d2, 1e-12)
        return a * jax.lax.rsqrt(a)

    ds = dist(src_all[...], srcT_blk[...], sqs_col[...], sqs_row[...])
    dt = dist(tgt_all[...], tgtT_blk[...], sqt_col[...], sqt_row[...])
    delta = ds - dt
    sc = jnp.maximum(1.0 - (delta ** 2) / (0.1 ** 2), 0.0)

    total = jnp.zeros((1, CB), jnp.float32)
    for k in range(8):
        w = sc[640 * k:640 * (k + 1), :]
        acc = w[0:8, :]
        for c in range(1, 80):
            acc = acc + w[8 * c:8 * c + 8, :]
        p = acc[0:4, :] + acc[4:8, :]
        q = p[0:2, :] + p[2:4, :]
        r = q[0:1, :] + q[1:2, :]
        total = total + r
    out_ref[...] = total


_counts_call = pl.pallas_call(
    _counts_kernel,
    grid=(NP // CB,),
    in_specs=[pl.BlockSpec((NP, 3), lambda b: (0, 0)),
              pl.BlockSpec((NP, 3), lambda b: (0, 0)),
              pl.BlockSpec((3, CB), lambda b: (0, b)),
              pl.BlockSpec((3, CB), lambda b: (0, b)),
              pl.BlockSpec((NP, 1), lambda b: (0, 0)),
              pl.BlockSpec((NP, 1), lambda b: (0, 0)),
              pl.BlockSpec((1, CB), lambda b: (0, b)),
              pl.BlockSpec((1, CB), lambda b: (0, b))],
    out_specs=pl.BlockSpec((1, CB), lambda b: (0, b)),
    out_shape=jax.ShapeDtypeStruct((1, NP), jnp.float32),
)


def _select_kernel(ccol_ref, crow_ref, out_ref):
    crow = crow_ref[...]
    jidx = jax.lax.broadcasted_iota(jnp.int32, (1, N), 1)
    kio = jax.lax.broadcasted_iota(jnp.int32, (1, K), 1)
    acc = jnp.zeros((1, K), jnp.int32)
    for c0 in range(0, N, SEL_CH):
        ci = ccol_ref[pl.ds(c0, SEL_CH), :]
        iidx = jax.lax.broadcasted_iota(jnp.int32, (SEL_CH, 1), 0) + c0
        beat = (crow > ci) | ((crow == ci) & (jidx < iidx))
        rank = jnp.sum(beat.astype(jnp.int32), axis=1)[:, None]
        hit = jnp.where(rank == kio, iidx, 0)
        acc = acc + jnp.sum(hit, axis=0)[None, :]
    out_ref[...] = acc


_select_call = pl.pallas_call(
    _select_kernel,
    in_specs=[pl.BlockSpec((N, 1), lambda: (0, 0)),
              pl.BlockSpec((1, N), lambda: (0, 0))],
    out_specs=pl.BlockSpec((1, K), lambda: (0, 0)),
    out_shape=jax.ShapeDtypeStruct((1, K), jnp.int32),
)


def kernel(src_corr_points, tgt_corr_points):
    src = src_corr_points.astype(jnp.float32)
    tgt = tgt_corr_points.astype(jnp.float32)
    pad_s = jnp.full((NP - N, 3), 1e6, jnp.float32)
    pad_t = jnp.full((NP - N, 3), 3e6, jnp.float32)
    srcp = jnp.concatenate([src, pad_s], axis=0)
    tgtp = jnp.concatenate([tgt, pad_t], axis=0)
    sqs = jnp.sum(srcp * srcp, axis=-1)
    sqt = jnp.sum(tgtp * tgtp, axis=-1)

    counts = _counts_call(srcp, tgtp, srcp.T, tgtp.T,
                          sqs[:, None], sqt[:, None],
                          sqs[None, :], sqt[None, :])
    counts = counts[:, 0:N]
    idx = _select_call(counts.T, counts)
    return idx[0]


# speedup vs baseline: 1.4395x; 1.0104x over previous
import jax
import jax.numpy as jnp
from jax.experimental import pallas as pl

N = 5000
NP = 5120
CB = 1024
K = 1500
SEL_CH = 500


def _counts_kernel(src_all, tgt_all, srcT_blk, tgtT_blk,
                   sqs_col, sqt_col, sqs_row, sqt_row, out_ref):
    def dist(pts, ptsT, sq_col, sq_row):
        g = jax.lax.dot_general(pts, ptsT, (((1,), (0,)), ((), ())),
                                preferred_element_type=jnp.float32)
        d2 = (sq_col + sq_row) - 2.0 * g
        a = jnp.maximum(d2, 1e-12)
        return a * jax.lax.rsqrt(a)

    ds = dist(src_all[...], srcT_blk[...], sqs_col[...], sqs_row[...])
    dt = dist(tgt_all[...], tgtT_blk[...], sqt_col[...], sqt_row[...])
    delta = ds - dt
    sc = jnp.maximum(1.0 - (delta ** 2) / (0.1 ** 2), 0.0)

    total = jnp.zeros((1, CB), jnp.float32)
    for k in range(8):
        w = sc[640 * k:640 * (k + 1), :]
        acc = w[0:8, :]
        for c in range(1, 80):
            acc = acc + w[8 * c:8 * c + 8, :]
        p = acc[0:4, :] + acc[4:8, :]
        q = p[0:2, :] + p[2:4, :]
        r = q[0:1, :] + q[1:2, :]
        total = total + r
    out_ref[...] = total


_counts_call = pl.pallas_call(
    _counts_kernel,
    grid=(NP // CB,),
    in_specs=[pl.BlockSpec((NP, 3), lambda b: (0, 0)),
              pl.BlockSpec((NP, 3), lambda b: (0, 0)),
              pl.BlockSpec((3, CB), lambda b: (0, b)),
              pl.BlockSpec((3, CB), lambda b: (0, b)),
              pl.BlockSpec((NP, 1), lambda b: (0, 0)),
              pl.BlockSpec((NP, 1), lambda b: (0, 0)),
              pl.BlockSpec((1, CB), lambda b: (0, b)),
              pl.BlockSpec((1, CB), lambda b: (0, b))],
    out_specs=pl.BlockSpec((1, CB), lambda b: (0, b)),
    out_shape=jax.ShapeDtypeStruct((1, NP), jnp.float32),
)


def _select_kernel(ccol_ref, crow_ref, out_ref):
    crow = crow_ref[...]
    jidx = jax.lax.broadcasted_iota(jnp.int32, (1, N), 1)
    kio = jax.lax.broadcasted_iota(jnp.int32, (1, K), 1)
    acc = jnp.zeros((1, K), jnp.int32)
    for c0 in range(0, N, SEL_CH):
        ci = ccol_ref[pl.ds(c0, SEL_CH), :]
        iidx = jax.lax.broadcasted_iota(jnp.int32, (SEL_CH, 1), 0) + c0
        beat = (crow > ci) | ((crow == ci) & (jidx < iidx))
        rank = jnp.sum(beat.astype(jnp.int32), axis=1)[:, None]
        hit = jnp.where(rank == kio, iidx, 0)
        acc = acc + jnp.sum(hit, axis=0)[None, :]
    out_ref[...] = acc


_select_call = pl.pallas_call(
    _select_kernel,
    in_specs=[pl.BlockSpec((N, 1), lambda: (0, 0)),
              pl.BlockSpec((1, N), lambda: (0, 0))],
    out_specs=pl.BlockSpec((1, K), lambda: (0, 0)),
    out_shape=jax.ShapeDtypeStruct((1, K), jnp.int32),
)


def kernel(src_corr_points, tgt_corr_points):
    src = src_corr_points.astype(jnp.float32)
    tgt = tgt_corr_points.astype(jnp.float32)
    pad_s = jnp.full((NP - N, 3), 1e6, jnp.float32)
    pad_t = jnp.full((NP - N, 3), 3e6, jnp.float32)
    srcp = jnp.concatenate([src, pad_s], axis=0)
    tgtp = jnp.concatenate([tgt, pad_t], axis=0)
    sqs = jnp.sum(srcp * srcp, axis=-1)
    sqt = jnp.sum(tgtp * tgtp, axis=-1)

    counts = _counts_call(srcp, tgtp, srcp.T, tgtp.T,
                          sqs[:, None], sqt[:, None],
                          sqs[None, :], sqt[None, :])
    counts = counts[:, 0:N]
    idx = _select_call(counts.T, counts)
    return idx[0]


# speedup vs baseline: 1.4573x; 1.0124x over previous
import jax
import jax.numpy as jnp
from jax.experimental import pallas as pl

N = 5000
NP = 5120
CB = 1024
K = 1500
SEL_CH = 500


def _counts_kernel(src_all, tgt_all, srcT_blk, tgtT_blk,
                   sqs_col, sqt_col, sqs_row, sqt_row, out_ref):
    def dist(pts, ptsT, sq_col, sq_row):
        g = jax.lax.dot_general(pts, ptsT, (((1,), (0,)), ((), ())),
                                preferred_element_type=jnp.float32)
        d2 = (sq_col + sq_row) - 2.0 * g
        a = jnp.maximum(d2, 1e-12)
        return a * jax.lax.rsqrt(a)

    ds = dist(src_all[...], srcT_blk[...], sqs_col[...], sqs_row[...])
    dt = dist(tgt_all[...], tgtT_blk[...], sqt_col[...], sqt_row[...])
    delta = ds - dt
    sc = jnp.maximum(1.0 - (delta ** 2) / (0.1 ** 2), 0.0)

    total = jnp.zeros((1, CB), jnp.float32)
    for k in range(8):
        w = sc[640 * k:640 * (k + 1), :]
        acc = w[0:8, :]
        for c in range(1, 80):
            acc = acc + w[8 * c:8 * c + 8, :]
        p = acc[0:4, :] + acc[4:8, :]
        q = p[0:2, :] + p[2:4, :]
        r = q[0:1, :] + q[1:2, :]
        total = total + r
    out_ref[...] = total


_counts_call = pl.pallas_call(
    _counts_kernel,
    grid=(NP // CB,),
    in_specs=[pl.BlockSpec((NP, 3), lambda b: (0, 0)),
              pl.BlockSpec((NP, 3), lambda b: (0, 0)),
              pl.BlockSpec((3, CB), lambda b: (0, b)),
              pl.BlockSpec((3, CB), lambda b: (0, b)),
              pl.BlockSpec((NP, 1), lambda b: (0, 0)),
              pl.BlockSpec((NP, 1), lambda b: (0, 0)),
              pl.BlockSpec((1, CB), lambda b: (0, b)),
              pl.BlockSpec((1, CB), lambda b: (0, b))],
    out_specs=pl.BlockSpec((1, CB), lambda b: (0, b)),
    out_shape=jax.ShapeDtypeStruct((1, NP), jnp.float32),
)


def _select_kernel(ccol_ref, crow_ref, out_ref):
    crow = crow_ref[...]
    jidx = jax.lax.broadcasted_iota(jnp.int32, (1, N), 1)
    kio = jax.lax.broadcasted_iota(jnp.int32, (1, K), 1).astype(jnp.float32)
    ones = jnp.ones((N, 1), jnp.float32)
    acc = jnp.zeros((1, K), jnp.int32)
    for c0 in range(0, N, SEL_CH):
        ci = ccol_ref[pl.ds(c0, SEL_CH), :]
        iidx = jax.lax.broadcasted_iota(jnp.int32, (SEL_CH, 1), 0) + c0
        beat = (crow > ci) | ((crow == ci) & (jidx < iidx))
        beat_f = jnp.where(beat, 1.0, 0.0)
        rank = jax.lax.dot_general(beat_f, ones, (((1,), (0,)), ((), ())),
                                   preferred_element_type=jnp.float32)
        hit = jnp.where(rank == kio, iidx, 0)
        acc = acc + jnp.sum(hit, axis=0)[None, :]
    out_ref[...] = acc


_select_call = pl.pallas_call(
    _select_kernel,
    in_specs=[pl.BlockSpec((N, 1), lambda: (0, 0)),
              pl.BlockSpec((1, N), lambda: (0, 0))],
    out_specs=pl.BlockSpec((1, K), lambda: (0, 0)),
    out_shape=jax.ShapeDtypeStruct((1, K), jnp.int32),
)


def kernel(src_corr_points, tgt_corr_points):
    src = src_corr_points.astype(jnp.float32)
    tgt = tgt_corr_points.astype(jnp.float32)
    pad_s = jnp.full((NP - N, 3), 1e6, jnp.float32)
    pad_t = jnp.full((NP - N, 3), 3e6, jnp.float32)
    srcp = jnp.concatenate([src, pad_s], axis=0)
    tgtp = jnp.concatenate([tgt, pad_t], axis=0)
    sqs = jnp.sum(srcp * srcp, axis=-1)
    sqt = jnp.sum(tgtp * tgtp, axis=-1)

    counts = _counts_call(srcp, tgtp, srcp.T, tgtp.T,
                          sqs[:, None], sqt[:, None],
                          sqs[None, :], sqt[None, :])
    counts = counts[:, 0:N]
    idx = _select_call(counts.T, counts)
    return idx[0]
